# baseline (speedup 1.0000x reference)
import jax
import jax.numpy as jnp
from jax import lax
from jax.experimental import pallas as pl
from jax.experimental.pallas import tpu as pltpu
from jax.experimental.pallas import tpu_sc as plsc

_VOCAB = 100000
_DIM = 64
_B = 16384
_L = 10
_NEG = 5
_T = _NEG + 1
_NC = 2
_NS = 16
_NW = _NC * _NS
_BPW = _B // _NW
_CH = 64
_NCHUNK = _BPW // _CH
_LANES = 16
_DC = _DIM // _LANES
_GROUPS = _CH // _LANES
_Y2COLS = 256
_Y2ROWS = _T * _B * _LANES // _Y2COLS
_BROWS = _B * _LANES // _Y2COLS
_CIDX = _CH * _L
_TIDX = _CH * _T


def _cbow_sc_body(combo_hbm, emb0_hbm, emb1_hbm, y_hbm,
                  idx2d, ctx_idx, tgt_idx, ctx_rows, tgt_rows, y_v, sem_g):
    wid = lax.axis_index("s") * _NC + lax.axis_index("c")
    base = wid * _BPW
    lane = lax.broadcasted_iota(jnp.int32, (_LANES,), 0)
    ctx_mask = lane < _L
    tgt_mask = lane < _T

    def chunk_body(c, carry):
        cb = base + c * _CH
        pltpu.sync_copy(combo_hbm.at[pl.ds(cb, _CH), pl.ds(0, 2 * _LANES)],
                        idx2d)
        for e in range(_CH):
            ctx_idx[pl.ds(e * _L, _LANES)] = idx2d[e, pl.ds(0, _LANES)]
            tgt_idx[pl.ds(e * _T, _LANES)] = idx2d[e, pl.ds(_LANES, _LANES)]
        handles = []
        for j in range(_CIDX // 128):
            handles.append(pltpu.async_copy(
                emb0_hbm.at[ctx_idx.at[pl.ds(j * 128, 128)]],
                ctx_rows.at[pl.ds(j * 128, 128)], sem_g))
        for j in range(_TIDX // 128):
            handles.append(pltpu.async_copy(
                emb1_hbm.at[tgt_idx.at[pl.ds(j * 128, 128)]],
                tgt_rows.at[pl.ds(j * 128, 128)], sem_g))
        for h in handles:
            h.wait()

        def group(g, carry2):
            for p in range(_LANES):
                e = g * _LANES + p
                csum = []
                for k in range(_DC):
                    s = ctx_rows[e * _L, pl.ds(k * _LANES, _LANES)]
                    for j in range(1, _L):
                        s = s + ctx_rows[e * _L + j,
                                         pl.ds(k * _LANES, _LANES)]
                    csum.append(s)
                for r in range(_T):
                    v = csum[0] * tgt_rows[e * _T + r, pl.ds(0, _LANES)]
                    for k in range(1, _DC):
                        v = v + csum[k] * tgt_rows[e * _T + r,
                                                   pl.ds(k * _LANES, _LANES)]
                    y_v[r, g, pl.ds(p * _LANES, _LANES)] = v
            return carry2

        lax.fori_loop(0, _GROUPS, group, 0)
        for r in range(_T):
            pltpu.sync_copy(
                y_v.at[r],
                y_hbm.at[pl.ds(r * _BROWS + (cb // _LANES), _GROUPS)])
        return carry

    lax.fori_loop(0, _NCHUNK, chunk_body, 0)


def _loss_tc_body(y2_ref, lens_ref, o_ref):
    y2 = y2_ref[...]
    seg = (lax.broadcasted_iota(jnp.int32, (_Y2COLS, _LANES), 0) // _LANES
           == lax.broadcasted_iota(jnp.int32, (_Y2COLS, _LANES), 1))
    x = jnp.dot(y2, seg.astype(jnp.float32),
                preferred_element_type=jnp.float32)
    x3 = x.reshape(_T, _BROWS, _LANES) / lens_ref[...][None, :, :]
    sgn = jnp.where(
        lax.broadcasted_iota(jnp.int32, (_T, _BROWS, _LANES), 0) == 0,
        1.0, -1.0)
    terms = -jax.nn.log_sigmoid(sgn * jnp.clip(x3, -10.0, 10.0))
    o_ref[...] = jnp.sum(terms)[None, None]


@jax.jit
def _cbow(combo, lens2, emb0_weight, emb1_weight):
    mesh = plsc.VectorSubcoreMesh(core_axis_name="c", subcore_axis_name="s")
    y = pl.kernel(
        _cbow_sc_body,
        mesh=mesh,
        compiler_params=pltpu.CompilerParams(use_tc_tiling_on_sc=False),
        out_type=jax.ShapeDtypeStruct((_Y2ROWS, _Y2COLS), jnp.float32),
        scratch_types=[
            pltpu.VMEM((_CH, 2 * _LANES), jnp.int32),
            pltpu.VMEM((_CIDX + _LANES, ), jnp.int32),
            pltpu.VMEM((_TIDX + _LANES, ), jnp.int32),
            pltpu.VMEM((_CIDX, _DIM), jnp.float32),
            pltpu.VMEM((_TIDX, _DIM), jnp.float32),
            pltpu.VMEM((_T, _GROUPS, _Y2COLS), jnp.float32),
            pltpu.SemaphoreType.DMA,
        ],
    )(combo, emb0_weight, emb1_weight)
    o = pl.pallas_call(
        _loss_tc_body,
        out_shape=jax.ShapeDtypeStruct((1, 1), jnp.float32),
    )(y, lens2)
    return o[0, 0]


def kernel(word_idx, ctx_inds, ctx_lens, neg_inds, emb0_weight, emb1_weight):
    combo = jnp.concatenate(
        [
            ctx_inds.astype(jnp.int32),
            jnp.zeros((_B, _LANES - _L), jnp.int32),
            word_idx.astype(jnp.int32)[:, None],
            neg_inds.astype(jnp.int32),
            jnp.zeros((_B, 128 - _LANES - _T), jnp.int32),
        ],
        axis=1,
    )
    lens2 = ctx_lens.astype(jnp.float32).reshape(_BROWS, _LANES)
    return _cbow(combo, lens2, emb0_weight, emb1_weight)


# speedup vs baseline: 1.0636x; 1.0636x over previous
import jax
import jax.numpy as jnp
from jax import lax
from jax.experimental import pallas as pl
from jax.experimental.pallas import tpu as pltpu
from jax.experimental.pallas import tpu_sc as plsc

_VOCAB = 100000
_DIM = 64
_B = 16384
_L = 10
_NEG = 5
_T = _NEG + 1
_NC = 2
_NS = 16
_NW = _NC * _NS
_BPW = _B // _NW
_CH = 64
_NCHUNK = _BPW // _CH
_LANES = 16
_DC = _DIM // _LANES
_GROUPS = _CH // _LANES
_Y2COLS = 256
_Y2ROWS = _T * _B * _LANES // _Y2COLS
_BROWS = _B * _LANES // _Y2COLS
_CIDX = _CH * _L
_NIDX = _CH * _NEG


def _cbow_sc_body(ctx_hbm, word_hbm, neg_hbm, emb0_hbm, emb1_hbm, y_hbm,
                  ctx2d, neg2d, word_st, ctx_idx, neg_idx,
                  ctx_rows, word_rows, neg_rows, y_v, sem_g):
    wid = lax.axis_index("s") * _NC + lax.axis_index("c")
    base = wid * _BPW

    def chunk_body(c, carry):
        cb = base + c * _CH
        pltpu.sync_copy(ctx_hbm.at[pl.ds(cb, _CH), pl.ds(0, _LANES)], ctx2d)
        pltpu.sync_copy(neg_hbm.at[pl.ds(cb, _CH), pl.ds(0, _LANES)], neg2d)
        pltpu.sync_copy(word_hbm.at[pl.ds(pl.multiple_of(cb, 8), _CH)],
                        word_st)
        for e in range(_CH):
            ctx_idx[pl.ds(e * _L, _LANES)] = ctx2d[e, pl.ds(0, _LANES)]
            neg_idx[pl.ds(e * _NEG, _LANES)] = neg2d[e, pl.ds(0, _LANES)]
        handles = []
        for j in range(_CIDX // 128):
            handles.append(pltpu.async_copy(
                emb0_hbm.at[ctx_idx.at[pl.ds(j * 128, 128)]],
                ctx_rows.at[pl.ds(j * 128, 128)], sem_g))
        handles.append(pltpu.async_copy(
            emb1_hbm.at[word_st], word_rows, sem_g))
        for j in range(_NIDX // 64):
            handles.append(pltpu.async_copy(
                emb1_hbm.at[neg_idx.at[pl.ds(j * 64, 64)]],
                neg_rows.at[pl.ds(j * 64, 64)], sem_g))
        for h in handles:
            h.wait()

        def group(g, carry2):
            for p in range(_LANES):
                e = g * _LANES + p
                csum = []
                for k in range(_DC):
                    s = ctx_rows[e * _L, pl.ds(k * _LANES, _LANES)]
                    for j in range(1, _L):
                        s = s + ctx_rows[e * _L + j,
                                         pl.ds(k * _LANES, _LANES)]
                    csum.append(s)
                v = csum[0] * word_rows[e, pl.ds(0, _LANES)]
                for k in range(1, _DC):
                    v = v + csum[k] * word_rows[e, pl.ds(k * _LANES, _LANES)]
                y_v[0, g, pl.ds(p * _LANES, _LANES)] = v
                for r in range(_NEG):
                    v = csum[0] * neg_rows[e * _NEG + r, pl.ds(0, _LANES)]
                    for k in range(1, _DC):
                        v = v + csum[k] * neg_rows[e * _NEG + r,
                                                   pl.ds(k * _LANES, _LANES)]
                    y_v[1 + r, g, pl.ds(p * _LANES, _LANES)] = v
            return carry2

        lax.fori_loop(0, _GROUPS, group, 0)
        for r in range(_T):
            pltpu.sync_copy(
                y_v.at[r],
                y_hbm.at[pl.ds(r * _BROWS + (cb // _LANES), _GROUPS)])
        return carry

    lax.fori_loop(0, _NCHUNK, chunk_body, 0)


def _loss_tc_body(y2_ref, lens_ref, o_ref):
    y2 = y2_ref[...]
    seg = (lax.broadcasted_iota(jnp.int32, (_Y2COLS, _LANES), 0) // _LANES
           == lax.broadcasted_iota(jnp.int32, (_Y2COLS, _LANES), 1))
    x = jnp.dot(y2, seg.astype(jnp.float32),
                preferred_element_type=jnp.float32)
    x3 = x.reshape(_T, _BROWS, _LANES) / lens_ref[...][None, :, :]
    sgn = jnp.where(
        lax.broadcasted_iota(jnp.int32, (_T, _BROWS, _LANES), 0) == 0,
        1.0, -1.0)
    terms = -jax.nn.log_sigmoid(sgn * jnp.clip(x3, -10.0, 10.0))
    o_ref[...] = jnp.sum(terms)[None, None]


@jax.jit
def _cbow(ctx_pad, word_idx, neg_pad, lens2, emb0_weight, emb1_weight):
    mesh = plsc.VectorSubcoreMesh(core_axis_name="c", subcore_axis_name="s")
    y = pl.kernel(
        _cbow_sc_body,
        mesh=mesh,
        compiler_params=pltpu.CompilerParams(use_tc_tiling_on_sc=False),
        out_type=jax.ShapeDtypeStruct((_Y2ROWS, _Y2COLS), jnp.float32),
        scratch_types=[
            pltpu.VMEM((_CH, _LANES), jnp.int32),
            pltpu.VMEM((_CH, _LANES), jnp.int32),
            pltpu.VMEM((_CH,), jnp.int32),
            pltpu.VMEM((_CIDX + _LANES,), jnp.int32),
            pltpu.VMEM((_NIDX + _LANES,), jnp.int32),
            pltpu.VMEM((_CIDX, _DIM), jnp.float32),
            pltpu.VMEM((_CH, _DIM), jnp.float32),
            pltpu.VMEM((_NIDX, _DIM), jnp.float32),
            pltpu.VMEM((_T, _GROUPS, _Y2COLS), jnp.float32),
            pltpu.SemaphoreType.DMA,
        ],
    )(ctx_pad, word_idx, neg_pad, emb0_weight, emb1_weight)
    o = pl.pallas_call(
        _loss_tc_body,
        out_shape=jax.ShapeDtypeStruct((1, 1), jnp.float32),
    )(y, lens2)
    return o[0, 0]


def kernel(word_idx, ctx_inds, ctx_lens, neg_inds, emb0_weight, emb1_weight):
    ctx_pad = jnp.pad(ctx_inds.astype(jnp.int32), ((0, 0), (0, 128 - _L)))
    neg_pad = jnp.pad(neg_inds.astype(jnp.int32), ((0, 0), (0, 128 - _NEG)))
    lens2 = ctx_lens.astype(jnp.float32).reshape(_BROWS, _LANES)
    return _cbow(ctx_pad, word_idx.astype(jnp.int32), neg_pad, lens2,
                 emb0_weight, emb1_weight)
